# baseline (speedup 1.0000x reference)
import functools

import jax
import jax.numpy as jnp
from jax import lax
from jax.experimental import pallas as pl
from jax.experimental.pallas import tpu as pltpu
from jax.experimental.pallas import tpu_sc as plsc

N = 10000
E = 320000
D = 128

NC = 2
NS = 16
NW = NC * NS
LANES = 16

EPT = E // NW
K = 40
NCHUNK = EPT // K
NBUF = 5
RPT = 624
TAIL = N - NS * RPT

_mesh = plsc.VectorSubcoreMesh(core_axis_name="c", subcore_axis_name="s")
_sc_params = pltpu.CompilerParams(needs_layout_passes=False)


@functools.partial(
    pl.kernel,
    out_type=jax.ShapeDtypeStruct((NW, N), jnp.float32),
    mesh=_mesh,
    compiler_params=_sc_params,
    scratch_types=[
        pltpu.VMEM((EPT,), jnp.int32),
        pltpu.VMEM((EPT,), jnp.float32),
        pltpu.VMEM((N,), jnp.float32),
    ],
)
def _deg_kernel(dst_hbm, ew_hbm, out_hbm, dst_v, ew_v, deg_v):
    cid = lax.axis_index("c")
    sid = lax.axis_index("s")
    wid = sid * NC + cid
    base = wid * EPT
    pltpu.sync_copy(dst_hbm.at[pl.ds(base, EPT)], dst_v)
    pltpu.sync_copy(ew_hbm.at[pl.ds(base, EPT)], ew_v)

    def zero_body(i, carry):
        deg_v[pl.ds(i * LANES, LANES)] = jnp.zeros((LANES,), jnp.float32)
        return carry

    lax.fori_loop(0, N // LANES, zero_body, 0)

    def body(i, carry):
        idx = dst_v[pl.ds(i * LANES, LANES)]
        w = ew_v[pl.ds(i * LANES, LANES)]
        plsc.addupdate_scatter(deg_v, [idx], w)
        return carry

    lax.fori_loop(0, EPT // LANES, body, 0)
    pltpu.sync_copy(deg_v, out_hbm.at[wid])


_agg_scratch = (
    [pltpu.VMEM((K, D), jnp.float32) for _ in range(NBUF)]
    + [pltpu.VMEM((6, K), jnp.int32) for _ in range(NBUF)]
    + [
        pltpu.VMEM_SHARED((N, D), jnp.float32),
        pltpu.SemaphoreType.DMA((NBUF,)),
        pltpu.SemaphoreType.DMA((NBUF,)),
        pltpu.SemaphoreType.DMA((NBUF,)),
    ]
)


@functools.partial(
    pl.kernel,
    out_type=jax.ShapeDtypeStruct((NC, N, D), jnp.float32),
    mesh=_mesh,
    compiler_params=_sc_params,
    scratch_types=_agg_scratch,
)
def _agg_kernel(meta_hbm, h2_hbm, out_hbm, *scr):
    rows = scr[0:NBUF]
    meta = scr[NBUF:2 * NBUF]
    acc_sh, gsem, ssem, isem = scr[2 * NBUF:]
    cid = lax.axis_index("c")
    sid = lax.axis_index("s")
    wid = sid * NC + cid

    def zb(i, carry):
        for cc in range(D // LANES):
            rows[0][i, pl.ds(cc * LANES, LANES)] = jnp.zeros(
                (LANES,), jnp.float32)
        return carry

    lax.fori_loop(0, K, zb, 0)
    r0 = sid * RPT
    for b in range(RPT // K):
        pltpu.sync_copy(rows[0], acc_sh.at[pl.ds(r0 + b * K, K)])
    pltpu.sync_copy(rows[0].at[pl.ds(0, RPT % K)],
                    acc_sh.at[pl.ds(r0 + (RPT // K) * K, RPT % K)])

    @pl.when(sid == NS - 1)
    def _zero_tail():
        pltpu.sync_copy(rows[0].at[pl.ds(0, TAIL)],
                        acc_sh.at[pl.ds(NS * RPT, TAIL)])

    plsc.subcore_barrier()

    def issue_idx_pair(m, s):
        pltpu.async_copy(meta_hbm.at[m * NW + wid], meta[s], isem.at[s])

    def wait_idx_pair(s):
        pltpu.make_async_copy(meta_hbm.at[0], meta[s], isem.at[s]).wait()

    def issue_gather(b, s, par):
        pltpu.async_copy(h2_hbm.at[meta[s].at[par]], rows[b], gsem.at[b])

    def wait_gather(b, s, par):
        pltpu.make_async_copy(h2_hbm.at[meta[s].at[par]], rows[b],
                              gsem.at[b]).wait()

    def issue_scatter(b, s, par):
        pltpu.async_copy(rows[b], acc_sh.at[meta[s].at[2 + par]], ssem.at[b],
                         add=True)

    def wait_scatter(b, s, par):
        pltpu.make_async_copy(rows[b], acc_sh.at[meta[s].at[2 + par]],
                              ssem.at[b]).wait()

    zero_vec = jnp.zeros((LANES,), jnp.int32)

    def scale_rows(ps, par, b):
        row_vec = jnp.full((LANES,), 4 + par, jnp.int32)

        def scale(q, c2):
            e0 = q * 4
            for u in range(4):
                e = e0 + u
                s = plsc.bitcast(
                    plsc.load_gather(meta[ps], [row_vec, zero_vec + e]),
                    jnp.float32)
                for r in range(D // LANES):
                    rows[b][e, pl.ds(r * LANES, LANES)] = (
                        rows[b][e, pl.ds(r * LANES, LANES)] * s)
            return c2

        lax.fori_loop(0, K // 4, scale, 0)

    def do_chunk(j, u, swait, idx_wait, pref_g, pref_i):
        b = u % NBUF
        ps = (u // 2) % NBUF
        par = u % 2
        wait_gather(b, ps, par)
        scale_rows(ps, par, b)
        issue_scatter(b, ps, par)
        if swait:
            b3 = (u + 3) % 10
            wait_scatter(b3 % NBUF, (b3 // 2) % NBUF, b3 % 2)
        if pref_g:
            u2 = (u + 2) % 10
            if idx_wait:
                wait_idx_pair((u2 // 2) % NBUF)
            issue_gather(u2 % NBUF, (u2 // 2) % NBUF, u2 % 2)
        if pref_i:
            issue_idx_pair((j // 2) + 2, ((u // 2) + 2) % NBUF)

    for m in range(3):
        issue_idx_pair(m, m)
    wait_idx_pair(0)
    issue_gather(0, 0, 0)
    issue_gather(1, 0, 1)

    for u in range(10):
        do_chunk(u, u, swait=(u >= 2), idx_wait=(u % 2 == 0),
                 pref_g=True, pref_i=(u % 2 == 0 and u >= 2))

    def body(p, carry):
        j0 = p * 10
        for u in range(10):
            do_chunk(j0 + u, u, swait=True, idx_wait=(u % 2 == 0),
                     pref_g=True, pref_i=(u % 2 == 0))
        return carry

    lax.fori_loop(1, NCHUNK // 10 - 1, body, 0)

    jt = NCHUNK - 10
    for u in range(10):
        j = jt + u
        do_chunk(j, u, swait=True, idx_wait=(u % 2 == 0 and j + 2 < NCHUNK),
                 pref_g=(j + 2 < NCHUNK),
                 pref_i=(u % 2 == 0 and j + 4 < NCHUNK))

    wait_scatter(3, 4, 0)
    wait_scatter(4, 4, 1)

    plsc.subcore_barrier()
    pltpu.sync_copy(acc_sh.at[pl.ds(r0, RPT)],
                    out_hbm.at[cid, pl.ds(r0, RPT)])

    @pl.when(sid == NS - 1)
    def _out_tail():
        pltpu.sync_copy(acc_sh.at[pl.ds(NS * RPT, TAIL)],
                        out_hbm.at[cid, pl.ds(NS * RPT, TAIL)])


def _prep_body(x_ref, wg_ref, wr_ref, wh_ref, degp_ref,
               h2_ref, gate_ref, res_ref):
    deg = jnp.sum(degp_ref[...], axis=0) + 2.0
    dinv = jnp.where(deg > 0, lax.rsqrt(jnp.maximum(deg, 1e-12)), 0.0)
    xb = x_ref[...]
    h = jnp.dot(xb, wg_ref[...], preferred_element_type=jnp.float32)
    h2_ref[...] = h * dinv[:, None]
    gate_ref[...] = jax.nn.sigmoid(
        jnp.dot(xb, wh_ref[...], preferred_element_type=jnp.float32))
    res_ref[...] = jnp.dot(xb, wr_ref[...], preferred_element_type=jnp.float32)


_prep_call = pl.pallas_call(
    _prep_body,
    out_shape=[
        jax.ShapeDtypeStruct((N, D), jnp.float32),
        jax.ShapeDtypeStruct((N, D), jnp.float32),
        jax.ShapeDtypeStruct((N, D), jnp.float32),
    ],
)


def _final_body(s_ref, degp_ref, h2_ref, gate_ref, res_ref, out_ref):
    deg = jnp.sum(degp_ref[...], axis=0) + 2.0
    dinv = jnp.where(deg > 0, lax.rsqrt(jnp.maximum(deg, 1e-12)), 0.0)
    h2 = h2_ref[...]
    ssum = s_ref[0] + s_ref[1] + 2.0 * h2
    gcn = dinv[:, None] * ssum
    gate = gate_ref[...]
    rep = gate * gcn + (1.0 - gate) * res_ref[...]
    out_ref[...] = jnp.where(rep >= 0, rep, 0.01 * rep)


_final_call = pl.pallas_call(
    _final_body,
    out_shape=jax.ShapeDtypeStruct((N, D), jnp.float32),
)


def kernel(x, edge_index, adj_weight, W_gcn, W_res, W_hw):
    src = edge_index[0]
    dst = edge_index[1]
    deg_parts = _deg_kernel(dst, adj_weight)
    h2, gate, res = _prep_call(x, W_gcn, W_res, W_hw, deg_parts)
    npair = E // (2 * K)
    meta = jnp.concatenate(
        [src.reshape(npair, 2, K), dst.reshape(npair, 2, K),
         lax.bitcast_convert_type(adj_weight, jnp.int32).reshape(npair, 2, K)],
        axis=1)
    s_parts = _agg_kernel(meta, h2)
    return _final_call(s_parts, deg_parts, h2, gate, res)


# speedup vs baseline: 1.1497x; 1.1497x over previous
import functools

import jax
import jax.numpy as jnp
from jax import lax
from jax.experimental import pallas as pl
from jax.experimental.pallas import tpu as pltpu
from jax.experimental.pallas import tpu_sc as plsc

N = 10000
E = 320000
D = 128

NC = 2
NS = 16
NW = NC * NS
LANES = 16

EPT = E // NW
K = 40
NCHUNK = EPT // K
NBUF = 5
RPT = 624
TAIL = N - NS * RPT

_mesh = plsc.VectorSubcoreMesh(core_axis_name="c", subcore_axis_name="s")
_sc_params = pltpu.CompilerParams(needs_layout_passes=False)


@functools.partial(
    pl.kernel,
    out_type=jax.ShapeDtypeStruct((NW, N), jnp.float32),
    mesh=_mesh,
    compiler_params=_sc_params,
    scratch_types=[
        pltpu.VMEM((EPT,), jnp.int32),
        pltpu.VMEM((EPT,), jnp.float32),
        pltpu.VMEM((N,), jnp.float32),
    ],
)
def _deg_kernel(dst_hbm, ew_hbm, out_hbm, dst_v, ew_v, deg_v):
    cid = lax.axis_index("c")
    sid = lax.axis_index("s")
    wid = sid * NC + cid
    base = wid * EPT
    pltpu.sync_copy(dst_hbm.at[pl.ds(base, EPT)], dst_v)
    pltpu.sync_copy(ew_hbm.at[pl.ds(base, EPT)], ew_v)

    def zero_body(i, carry):
        deg_v[pl.ds(i * LANES, LANES)] = jnp.zeros((LANES,), jnp.float32)
        return carry

    lax.fori_loop(0, N // LANES, zero_body, 0)

    def body(i, carry):
        idx = dst_v[pl.ds(i * LANES, LANES)]
        w = ew_v[pl.ds(i * LANES, LANES)]
        plsc.addupdate_scatter(deg_v, [idx], w)
        return carry

    lax.fori_loop(0, EPT // LANES, body, 0)
    pltpu.sync_copy(deg_v, out_hbm.at[wid])


_agg_scratch = (
    [pltpu.VMEM((K, D), jnp.float32) for _ in range(NBUF)]
    + [pltpu.VMEM((K,), jnp.int32) for _ in range(NBUF)]
    + [pltpu.VMEM((K,), jnp.int32) for _ in range(NBUF)]
    + [
        pltpu.VMEM((EPT + 16,), jnp.float32),
        pltpu.VMEM_SHARED((N, D), jnp.float32),
        pltpu.SemaphoreType.DMA((NBUF,)),
        pltpu.SemaphoreType.DMA((NBUF,)),
        pltpu.SemaphoreType.DMA((NBUF,)),
    ]
)


@functools.partial(
    pl.kernel,
    out_type=jax.ShapeDtypeStruct((NC, N, D), jnp.float32),
    mesh=_mesh,
    compiler_params=_sc_params,
    scratch_types=_agg_scratch,
)
def _agg_kernel(src_hbm, dst_hbm, ew_hbm, h2_hbm, out_hbm, *scr):
    rows = scr[0:NBUF]
    sidx = scr[NBUF:2 * NBUF]
    didx = scr[2 * NBUF:3 * NBUF]
    ew_v, acc_sh, gsem, ssem, isem = scr[3 * NBUF:]
    cid = lax.axis_index("c")
    sid = lax.axis_index("s")
    wid = sid * NC + cid
    base = wid * EPT

    pltpu.sync_copy(ew_hbm.at[pl.ds(base, EPT)], ew_v.at[pl.ds(0, EPT)])

    def zb(i, carry):
        for cc in range(D // LANES):
            rows[0][i, pl.ds(cc * LANES, LANES)] = jnp.zeros(
                (LANES,), jnp.float32)
        return carry

    lax.fori_loop(0, K, zb, 0)
    r0 = sid * RPT
    for b in range(RPT // K):
        pltpu.sync_copy(rows[0], acc_sh.at[pl.ds(r0 + b * K, K)])
    pltpu.sync_copy(rows[0].at[pl.ds(0, RPT % K)],
                    acc_sh.at[pl.ds(r0 + (RPT // K) * K, RPT % K)])

    @pl.when(sid == NS - 1)
    def _zero_tail():
        pltpu.sync_copy(rows[0].at[pl.ds(0, TAIL)],
                        acc_sh.at[pl.ds(NS * RPT, TAIL)])

    plsc.subcore_barrier()

    def issue_idx(j, s):
        off = base + j * K
        pltpu.async_copy(src_hbm.at[pl.ds(off, K)], sidx[s], isem.at[s])
        pltpu.async_copy(dst_hbm.at[pl.ds(off, K)], didx[s], isem.at[s])

    def wait_idx(s):
        pltpu.make_async_copy(src_hbm.at[pl.ds(0, K)], sidx[s],
                              isem.at[s]).wait()
        pltpu.make_async_copy(dst_hbm.at[pl.ds(0, K)], didx[s],
                              isem.at[s]).wait()

    def issue_gather(s):
        pltpu.async_copy(h2_hbm.at[sidx[s]], rows[s], gsem.at[s])

    def wait_gather(s):
        pltpu.make_async_copy(h2_hbm.at[sidx[s]], rows[s], gsem.at[s]).wait()

    def issue_scatter(s):
        pltpu.async_copy(rows[s], acc_sh.at[didx[s]], ssem.at[s], add=True)

    def wait_scatter(s):
        pltpu.make_async_copy(rows[s], acc_sh.at[didx[s]], ssem.at[s]).wait()

    def scale_rows(j, b):
        base_vec = jnp.full((LANES,), j * K, jnp.int32)

        def scale(q, c2):
            e0 = q * 4
            for u in range(4):
                e = e0 + u
                s = plsc.load_gather(ew_v, [base_vec + e])
                for r in range(D // LANES):
                    rows[b][e, pl.ds(r * LANES, LANES)] = (
                        rows[b][e, pl.ds(r * LANES, LANES)] * s)
            return c2

        lax.fori_loop(0, K // 4, scale, 0)

    def do_chunk(j, b, swait, pref_g, pref_i):
        wait_gather(b)
        scale_rows(j, b)
        issue_scatter(b)
        if swait:
            wait_scatter((b + 3) % NBUF)
        if pref_g:
            wait_idx((b + 2) % NBUF)
            issue_gather((b + 2) % NBUF)
        if pref_i:
            issue_idx(j + 3, (b + 3) % NBUF)

    for s in range(3):
        issue_idx(s, s)
    for s in range(2):
        wait_idx(s)
        issue_gather(s)
    for b in range(NBUF):
        do_chunk(b, b, swait=(b >= 2), pref_g=True, pref_i=True)

    def body(p, carry):
        j0 = p * NBUF
        for b in range(NBUF):
            do_chunk(j0 + b, b, swait=True, pref_g=True, pref_i=True)
        return carry

    lax.fori_loop(1, NCHUNK // NBUF - 1, body, 0)

    jt = NCHUNK - NBUF
    for b in range(NBUF):
        j = jt + b
        do_chunk(j, b, swait=True, pref_g=(j + 2 < NCHUNK),
                 pref_i=(j + 3 < NCHUNK))

    wait_scatter(3)
    wait_scatter(4)

    plsc.subcore_barrier()
    pltpu.sync_copy(acc_sh.at[pl.ds(r0, RPT)],
                    out_hbm.at[cid, pl.ds(r0, RPT)])

    @pl.when(sid == NS - 1)
    def _out_tail():
        pltpu.sync_copy(acc_sh.at[pl.ds(NS * RPT, TAIL)],
                        out_hbm.at[cid, pl.ds(NS * RPT, TAIL)])


def _prep_body(x_ref, wg_ref, wr_ref, wh_ref, degp_ref,
               h2_ref, gate_ref, res_ref):
    deg = jnp.sum(degp_ref[...], axis=0) + 2.0
    dinv = jnp.where(deg > 0, lax.rsqrt(jnp.maximum(deg, 1e-12)), 0.0)
    xb = x_ref[...]
    h = jnp.dot(xb, wg_ref[...], preferred_element_type=jnp.float32)
    h2_ref[...] = h * dinv[:, None]
    gate_ref[...] = jax.nn.sigmoid(
        jnp.dot(xb, wh_ref[...], preferred_element_type=jnp.float32))
    res_ref[...] = jnp.dot(xb, wr_ref[...], preferred_element_type=jnp.float32)


_prep_call = pl.pallas_call(
    _prep_body,
    out_shape=[
        jax.ShapeDtypeStruct((N, D), jnp.float32),
        jax.ShapeDtypeStruct((N, D), jnp.float32),
        jax.ShapeDtypeStruct((N, D), jnp.float32),
    ],
)


def _final_body(s_ref, degp_ref, h2_ref, gate_ref, res_ref, out_ref):
    deg = jnp.sum(degp_ref[...], axis=0) + 2.0
    dinv = jnp.where(deg > 0, lax.rsqrt(jnp.maximum(deg, 1e-12)), 0.0)
    h2 = h2_ref[...]
    ssum = s_ref[0] + s_ref[1] + 2.0 * h2
    gcn = dinv[:, None] * ssum
    gate = gate_ref[...]
    rep = gate * gcn + (1.0 - gate) * res_ref[...]
    out_ref[...] = jnp.where(rep >= 0, rep, 0.01 * rep)


_final_call = pl.pallas_call(
    _final_body,
    out_shape=jax.ShapeDtypeStruct((N, D), jnp.float32),
)


def kernel(x, edge_index, adj_weight, W_gcn, W_res, W_hw):
    src = edge_index[0]
    dst = edge_index[1]
    deg_parts = _deg_kernel(dst, adj_weight)
    h2, gate, res = _prep_call(x, W_gcn, W_res, W_hw, deg_parts)
    s_parts = _agg_kernel(src, dst, adj_weight, h2)
    return _final_call(s_parts, deg_parts, h2, gate, res)
